# Initial kernel scaffold; baseline (speedup 1.0000x reference)
import jax
import jax.numpy as jnp
from jax.experimental import pallas as pl


def kernel(lengths, maxlen, mask_pad):
    raise NotImplementedError("write your pallas kernel here")


# speedup vs baseline: 1.1015x; 1.1015x over previous
import functools

import jax
import jax.numpy as jnp
from jax import lax
from jax.experimental import pallas as pl
from jax.experimental.pallas import tpu as pltpu
from jax.experimental.pallas import tpu_sc as plsc

MAXLEN = 2048
BATCH = 16384
NC, NS, L = 2, 16, 16
NW = NC * NS
BPW = BATCH // NW
CHUNK = L
NCHUNK = BPW // CHUNK
NBUF = 2


def _body(len_hbm, table_hbm, out_hbm, len_v, bufs, sems):
    wid = lax.axis_index("s") * NC + lax.axis_index("c")
    row_base = wid * BPW

    pltpu.sync_copy(len_hbm.at[pl.ds(wid * NCHUNK, NCHUNK)], len_v)

    def idx_for(g):
        v = len_v[g] - 1
        v = jnp.where(v < 0, v + MAXLEN, v)
        return jnp.minimum(jnp.maximum(v, 0), MAXLEN - 1)

    copies = [None] * NBUF
    copies[0] = pltpu.make_async_copy(table_hbm.at[idx_for(0)], bufs[0], sems[0])
    copies[0].start()
    for g in range(NCHUNK):
        b = g % NBUF
        nb = (g + 1) % NBUF
        if g + 1 < NCHUNK:
            copies[nb] = pltpu.make_async_copy(
                table_hbm.at[idx_for(g + 1)], bufs[nb], sems[nb])
            copies[nb].start()
        copies[b].wait()
        pltpu.sync_copy(bufs[b], out_hbm.at[pl.ds(row_base + g * CHUNK, CHUNK)])


@jax.jit
def _make_pad_mask(len2, mask_pad):
    mesh = plsc.VectorSubcoreMesh(core_axis_name="c", subcore_axis_name="s")
    return pl.kernel(
        _body,
        out_type=jax.ShapeDtypeStruct((BATCH, MAXLEN), jnp.float32),
        mesh=mesh,
        scratch_types=[
            pltpu.VMEM((NCHUNK, L), jnp.int32),
            [pltpu.VMEM((CHUNK, MAXLEN), jnp.float32) for _ in range(NBUF)],
            [pltpu.SemaphoreType.DMA for _ in range(NBUF)],
        ],
    )(len2, mask_pad)


def kernel(lengths, maxlen, mask_pad):
    adj = jnp.asarray(maxlen).astype(jnp.int32) - mask_pad.shape[-1]
    len2 = (lengths.astype(jnp.int32) + adj).reshape(BATCH // L, L)
    return _make_pad_mask(len2, mask_pad)
